# Initial kernel scaffold; baseline (speedup 1.0000x reference)
import jax
import jax.numpy as jnp
from jax.experimental import pallas as pl


def kernel(x, edge_index, params):
    raise NotImplementedError("write your pallas kernel here")


# speedup vs baseline: 13.6056x; 13.6056x over previous
import functools

import jax
import jax.numpy as jnp
from jax import lax
from jax.experimental import pallas as pl
from jax.experimental.pallas import tpu as pltpu
from jax.experimental.pallas import tpu_sc as plsc

_NC = 2
_NS = 16
_NW = _NC * _NS


def _mm_bias_body(x_ref, w_ref, b_ref, o_ref):
    o_ref[...] = (
        jnp.dot(x_ref[...], w_ref[...], preferred_element_type=jnp.float32)
        + b_ref[...]
    )


def _mm_bias(x, w, b, bm=1000):
    m, k = x.shape
    n = w.shape[1]
    return pl.pallas_call(
        _mm_bias_body,
        grid=(m // bm,),
        in_specs=[
            pl.BlockSpec((bm, k), lambda i: (i, 0)),
            pl.BlockSpec((k, n), lambda i: (0, 0)),
            pl.BlockSpec((1, n), lambda i: (0, 0)),
        ],
        out_specs=pl.BlockSpec((bm, n), lambda i: (i, 0)),
        out_shape=jax.ShapeDtypeStruct((m, n), jnp.float32),
    )(x, w, b.reshape(1, -1))


def _mm2_bias_body(xa_ref, xb_ref, wa_ref, wb_ref, b_ref, o_ref):
    o_ref[...] = (
        jnp.dot(xa_ref[...], wa_ref[...], preferred_element_type=jnp.float32)
        + jnp.dot(xb_ref[...], wb_ref[...], preferred_element_type=jnp.float32)
        + b_ref[...]
    )


def _mm2_bias(xa, xb, wa, wb, b, bm=1000):
    m, ka = xa.shape
    n = wa.shape[1]
    kb = xb.shape[1]
    return pl.pallas_call(
        _mm2_bias_body,
        grid=(m // bm,),
        in_specs=[
            pl.BlockSpec((bm, ka), lambda i: (i, 0)),
            pl.BlockSpec((bm, kb), lambda i: (i, 0)),
            pl.BlockSpec((ka, n), lambda i: (0, 0)),
            pl.BlockSpec((kb, n), lambda i: (0, 0)),
            pl.BlockSpec((1, n), lambda i: (0, 0)),
        ],
        out_specs=pl.BlockSpec((bm, n), lambda i: (i, 0)),
        out_shape=jax.ShapeDtypeStruct((m, n), jnp.float32),
    )(xa, xb, wa, wb, b.reshape(1, -1))


def _lstm_pair_body(chunk, g, pf_ref, pb_ref, wf_ref, wb_ref,
                    of_ref, ob_ref, st_ref):
    i = pl.program_id(0)

    @pl.when(i == 0)
    def _():
        st_ref[...] = jnp.zeros_like(st_ref)

    wf = wf_ref[...]
    wb = wb_ref[...]

    def gates(gv, c):
        ig = jax.nn.sigmoid(gv[:, 0:128])
        fg = jax.nn.sigmoid(gv[:, 128:256])
        gg = jnp.tanh(gv[:, 256:384])
        og = jax.nn.sigmoid(gv[:, 384:512])
        c = fg * c + ig * gg
        h = og * jnp.tanh(c)
        return h, c

    def step(t, carry):
        hf, cf, hb, cb = carry
        gf = jnp.dot(hf, wf, preferred_element_type=jnp.float32) \
            + pf_ref[pl.ds(t, 1), :]
        gb = jnp.dot(hb, wb, preferred_element_type=jnp.float32) \
            + pb_ref[pl.ds(chunk - 1 - t, 1), :]
        hf, cf = gates(gf, cf)
        hb, cb = gates(gb, cb)
        of_ref[pl.ds(t, 1), :] = hf
        ob_ref[pl.ds(chunk - 1 - t, 1), :] = hb
        return hf, cf, hb, cb

    init = (st_ref[0:1, :], st_ref[1:2, :], st_ref[2:3, :], st_ref[3:4, :])
    hf, cf, hb, cb = lax.fori_loop(0, chunk, step, init)
    st_ref[0:1, :] = hf
    st_ref[1:2, :] = cf
    st_ref[2:3, :] = hb
    st_ref[3:4, :] = cb


def _lstm_pair(p_all, whh_f_t, whh_b_t, chunk=1000):
    t_len = p_all.shape[0]
    g = t_len // chunk
    body = functools.partial(_lstm_pair_body, chunk, g)
    return pl.pallas_call(
        body,
        grid=(g,),
        in_specs=[
            pl.BlockSpec((chunk, 512), lambda i: (i, 0)),
            pl.BlockSpec((chunk, 512), lambda i, _g=g: (_g - 1 - i, 1)),
            pl.BlockSpec((128, 512), lambda i: (0, 0)),
            pl.BlockSpec((128, 512), lambda i: (0, 0)),
        ],
        out_specs=[
            pl.BlockSpec((chunk, 128), lambda i: (i, 0)),
            pl.BlockSpec((chunk, 128), lambda i, _g=g: (_g - 1 - i, 0)),
        ],
        out_shape=[
            jax.ShapeDtypeStruct((t_len, 128), jnp.float32),
            jax.ShapeDtypeStruct((t_len, 128), jnp.float32),
        ],
        scratch_shapes=[pltpu.VMEM((8, 128), jnp.float32)],
    )(p_all, p_all, whh_f_t, whh_b_t)


def _dis_from_degp(dp_ref):
    deg = dp_ref[0] + dp_ref[1]
    return jnp.where(deg > 0, lax.rsqrt(jnp.maximum(deg, 1e-12)), 0.0)


def _u1_body(f1_ref, b1_ref, dp_ref, wa_ref, wb_ref, o_ref):
    dis = _dis_from_degp(dp_ref)
    h = (jnp.dot(f1_ref[...], wa_ref[...], preferred_element_type=jnp.float32)
         + jnp.dot(b1_ref[...], wb_ref[...], preferred_element_type=jnp.float32))
    o_ref[...] = h * dis


def _u1(f1, b1, degp, wa, wb):
    n = f1.shape[0]
    d = wa.shape[1]
    return pl.pallas_call(
        _u1_body,
        out_shape=jax.ShapeDtypeStruct((n, d), jnp.float32),
    )(f1, b1, degp, wa, wb)


def _bn_next_body(d_in, ap_ref, dp_ref, bg_ref, bnw_ref, bnb_ref, w_ref,
                  bo_ref, o_ref):
    dis = _dis_from_degp(dp_ref)
    gv = (ap_ref[0, :, 0:d_in] + ap_ref[1, :, 0:d_in]) * dis + bg_ref[...]
    m = jnp.mean(gv, axis=0, keepdims=True)
    dv = gv - m
    v = jnp.mean(dv * dv, axis=0, keepdims=True)
    gn = dv * lax.rsqrt(v + 1e-5) * bnw_ref[...] + bnb_ref[...]
    r = jnp.maximum(gn, 0.0)
    o_ref[...] = (
        jnp.dot(r, w_ref[...], preferred_element_type=jnp.float32)
        + bo_ref[...]
    )


def _bn_next(ap, degp, bg, bnw, bnb, w, bo, scale_dis):
    n = ap.shape[1]
    d = w.shape[1]
    d_in = bg.shape[0]
    if scale_dis:
        body = functools.partial(_bn_next_scaled_body, d_in)
    else:
        body = functools.partial(_bn_next_body, d_in)
    return pl.pallas_call(
        body,
        out_shape=jax.ShapeDtypeStruct((n, d), jnp.float32),
    )(ap, degp, bg.reshape(1, -1), bnw.reshape(1, -1), bnb.reshape(1, -1),
      w, bo.reshape(1, -1))


def _bn_next_scaled_body(d_in, ap_ref, dp_ref, bg_ref, bnw_ref, bnb_ref,
                         w_ref, bo_ref, o_ref):
    dis = _dis_from_degp(dp_ref)
    gv = (ap_ref[0, :, 0:d_in] + ap_ref[1, :, 0:d_in]) * dis + bg_ref[...]
    m = jnp.mean(gv, axis=0, keepdims=True)
    dv = gv - m
    v = jnp.mean(dv * dv, axis=0, keepdims=True)
    gn = dv * lax.rsqrt(v + 1e-5) * bnw_ref[...] + bnb_ref[...]
    r = jnp.maximum(gn, 0.0)
    o_ref[...] = (
        jnp.dot(r, w_ref[...], preferred_element_type=jnp.float32)
        + bo_ref[...]
    ) * dis


def _sc_mesh():
    return plsc.VectorSubcoreMesh(core_axis_name="c", subcore_axis_name="s",
                                  num_cores=_NC, num_subcores=_NS)


def _row_split(n_nodes):
    per = (n_nodes // (8 * _NS)) * 8
    tail = n_nodes - _NS * per
    return per, tail


def _zero_shared(sh_ref, z_hbm, n_nodes, sid):
    per, tail = _row_split(n_nodes)
    base = sid * per
    nfull = per // 128
    rem = per - nfull * 128
    for k in range(nfull):
        pltpu.sync_copy(z_hbm, sh_ref.at[pl.ds(base + k * 128, 128)])
    if rem:
        pltpu.sync_copy(z_hbm.at[pl.ds(0, rem)],
                        sh_ref.at[pl.ds(base + nfull * 128, rem)])
    if tail:
        @pl.when(sid == 0)
        def _():
            pltpu.sync_copy(z_hbm.at[pl.ds(0, tail)],
                            sh_ref.at[pl.ds(_NS * per, tail)])


def _writeback_shared(sh_ref, out_hbm, n_nodes, cid, sid):
    per, tail = _row_split(n_nodes)
    base = sid * per
    pltpu.sync_copy(sh_ref.at[pl.ds(base, per)],
                    out_hbm.at[cid, pl.ds(base, per)])
    if tail:
        @pl.when(sid == 0)
        def _():
            pltpu.sync_copy(sh_ref.at[pl.ds(_NS * per, tail)],
                            out_hbm.at[cid, pl.ds(_NS * per, tail)])


def _deg_body(n_pad, ept, ch, col_hbm, out_hbm, deg_v, col_v, sh_ref,
              buf_v, acc_v):
    cid = lax.axis_index("c")
    sid = lax.axis_index("s")
    wid = sid * _NC + cid
    zero16 = jnp.zeros((16,), jnp.float32)
    one16 = jnp.ones((16,), jnp.float32)

    def zero_body(i, _):
        deg_v[pl.ds(i * 16, 16)] = zero16
        return 0

    lax.fori_loop(0, n_pad // 16, zero_body, 0)

    nch = ept // ch

    def chunk_body(c, _):
        pltpu.sync_copy(col_hbm.at[pl.ds(wid * ept + c * ch, ch)], col_v)

        def it(k, _2):
            idx16 = col_v[pl.ds(k * 16, 16)]
            plsc.addupdate_scatter(deg_v, [idx16], one16)
            return 0

        lax.fori_loop(0, ch // 16, it, 0)
        return 0

    lax.fori_loop(0, nch, chunk_body, 0)

    pltpu.sync_copy(deg_v, sh_ref.at[sid])
    plsc.subcore_barrier()
    rows = n_pad // _NS
    base = sid * rows

    def zero2(i, _):
        acc_v[pl.ds(i * 16, 16)] = zero16
        return 0

    lax.fori_loop(0, rows // 16, zero2, 0)
    for j in range(_NS):
        pltpu.sync_copy(sh_ref.at[j, pl.ds(base, rows)], buf_v)

        def addb(i, _):
            acc_v[pl.ds(i * 16, 16)] = (acc_v[pl.ds(i * 16, 16)]
                                        + buf_v[pl.ds(i * 16, 16)])
            return 0

        lax.fori_loop(0, rows // 16, addb, 0)
    pltpu.sync_copy(acc_v, out_hbm.at[cid, pl.ds(base, rows)])


def _sc_degree(col, n_nodes):
    e = col.shape[0]
    ept = e // _NW
    ch = 4000
    n_pad = ((n_nodes + 2047) // 2048) * 2048
    body = functools.partial(_deg_body, n_pad, ept, ch)
    f = pl.kernel(
        body,
        out_type=jax.ShapeDtypeStruct((_NC, n_pad), jnp.float32),
        mesh=_sc_mesh(),
        compiler_params=pltpu.CompilerParams(needs_layout_passes=False),
        scratch_types=[
            pltpu.VMEM((n_pad,), jnp.float32),
            pltpu.VMEM((ch,), jnp.int32),
            pltpu.VMEM_SHARED((_NS, n_pad), jnp.float32),
            pltpu.VMEM((n_pad // _NS,), jnp.float32),
            pltpu.VMEM((n_pad // _NS,), jnp.float32),
        ],
    )
    return f(col)


def _agg_body(n_nodes, d, ept, u_hbm, row_hbm, col_hbm, z_hbm, out_hbm,
              sh_ref, rows_v, rows_r, ir_v, ic_v, ir_r, ic_r, sem):
    cid = lax.axis_index("c")
    sid = lax.axis_index("s")
    wid = sid * _NC + cid
    _zero_shared(sh_ref, z_hbm, n_nodes, sid)
    plsc.subcore_barrier()

    ebase = wid * ept
    nb = ept // 128
    rem = ept - nb * 128

    def body(b, _):
        pltpu.sync_copy(row_hbm.at[pl.ds(ebase + b * 128, 128)], ir_v)
        pltpu.sync_copy(col_hbm.at[pl.ds(ebase + b * 128, 128)], ic_v)
        pltpu.async_copy(u_hbm.at[ir_v], rows_v, sem).wait()
        pltpu.sync_copy(rows_v, sh_ref.at[ic_v], add=True)
        return 0

    lax.fori_loop(0, nb, body, 0)
    if rem:
        pltpu.sync_copy(row_hbm.at[pl.ds(ebase + nb * 128, rem)], ir_r)
        pltpu.sync_copy(col_hbm.at[pl.ds(ebase + nb * 128, rem)], ic_r)
        pltpu.async_copy(u_hbm.at[ir_r], rows_r, sem).wait()
        pltpu.sync_copy(rows_r, sh_ref.at[ic_r], add=True)
    plsc.subcore_barrier()
    _writeback_shared(sh_ref, out_hbm, n_nodes, cid, sid)


def _sc_aggregate(u, row, col):
    n_nodes, d = u.shape
    e = row.shape[0]
    ept = e // _NW
    z_blk = jnp.zeros((128, d), jnp.float32)
    body = functools.partial(_agg_body, n_nodes, d, ept)
    f = pl.kernel(
        body,
        out_type=jax.ShapeDtypeStruct((_NC, n_nodes, d), jnp.float32),
        mesh=_sc_mesh(),
        scratch_types=[
            pltpu.VMEM_SHARED((n_nodes, d), jnp.float32),
            pltpu.VMEM((128, d), jnp.float32),
            pltpu.VMEM((32, d), jnp.float32),
            pltpu.VMEM((128,), jnp.int32),
            pltpu.VMEM((128,), jnp.int32),
            pltpu.VMEM((32,), jnp.int32),
            pltpu.VMEM((32,), jnp.int32),
            pltpu.SemaphoreType.DMA,
        ],
    )
    return f(u, row, col, z_blk)


def _edge_out_body(n_nodes, ept, ch, zr_hbm, zc_hbm, row_hbm, col_hbm,
                   o0_hbm, o1_hbm, zr_v, zc_v, row_v, col_v, o0_v, o1_v):
    cid = lax.axis_index("c")
    sid = lax.axis_index("s")
    wid = sid * _NC + cid
    ebase = wid * ept
    pltpu.sync_copy(zr_hbm, zr_v)
    pltpu.sync_copy(zc_hbm, zc_v)

    nch = ept // ch

    def chunk_body(c, _):
        cbase = ebase + c * ch
        pltpu.sync_copy(row_hbm.at[pl.ds(cbase, ch)], row_v)
        pltpu.sync_copy(col_hbm.at[pl.ds(cbase, ch)], col_v)

        def it(k, _2):
            r2 = row_v[pl.ds(k * 16, 16)] * 2
            c2 = col_v[pl.ds(k * 16, 16)] * 2
            a0 = plsc.load_gather(zr_v, [r2])
            a1 = plsc.load_gather(zr_v, [r2 + 1])
            b0 = plsc.load_gather(zc_v, [c2])
            b1 = plsc.load_gather(zc_v, [c2 + 1])
            o0_v[pl.ds(k * 16, 16)] = a0 + b0
            o1_v[pl.ds(k * 16, 16)] = a1 + b1
            return 0

        lax.fori_loop(0, ch // 16, it, 0)
        pltpu.sync_copy(o0_v, o0_hbm.at[pl.ds(cbase, ch)])
        pltpu.sync_copy(o1_v, o1_hbm.at[pl.ds(cbase, ch)])
        return 0

    lax.fori_loop(0, nch, chunk_body, 0)


def _sc_edge_out(zr, zc, row, col):
    table_len = zr.shape[0]
    e = row.shape[0]
    ept = e // _NW
    ch = 4000
    body = functools.partial(_edge_out_body, table_len, ept, ch)
    f = pl.kernel(
        body,
        out_type=(
            jax.ShapeDtypeStruct((e,), jnp.float32),
            jax.ShapeDtypeStruct((e,), jnp.float32),
        ),
        mesh=_sc_mesh(),
        compiler_params=pltpu.CompilerParams(needs_layout_passes=False),
        scratch_types=[
            pltpu.VMEM((table_len,), jnp.float32),
            pltpu.VMEM((table_len,), jnp.float32),
            pltpu.VMEM((ch,), jnp.int32),
            pltpu.VMEM((ch,), jnp.int32),
            pltpu.VMEM((ch,), jnp.float32),
            pltpu.VMEM((ch,), jnp.float32),
        ],
    )
    return f(zr, zc, row, col)


def kernel(x, edge_index, params):
    p = params
    n_nodes = x.shape[0]
    row = edge_index[0]
    col = edge_index[1]

    w0 = jnp.concatenate([p['Wih_l0f'].T, p['Wih_l0b'].T], axis=1)
    bias0 = jnp.concatenate([p['bih_l0f'] + p['bhh_l0f'],
                             p['bih_l0b'] + p['bhh_l0b']])
    p0 = _mm_bias(x, w0, bias0)
    f0, b0 = _lstm_pair(p0, p['Whh_l0f'].T, p['Whh_l0b'].T)

    w1 = jnp.concatenate([p['Wih_l1f'].T, p['Wih_l1b'].T], axis=1)
    bias1 = jnp.concatenate([p['bih_l1f'] + p['bhh_l1f'],
                             p['bih_l1b'] + p['bhh_l1b']])
    p1 = _mm2_bias(f0, b0, w1[:128], w1[128:], bias1)
    f1, b1 = _lstm_pair(p1, p['Whh_l1f'].T, p['Whh_l1b'].T)

    degp = _sc_degree(col, n_nodes)
    degp = degp[:, :n_nodes, None]

    u1 = _u1(f1, b1, degp, p['W_g1'][:128], p['W_g1'][128:])
    ag1 = _sc_aggregate(u1, row, col)
    wg2p = jnp.zeros((128, 128), jnp.float32).at[:, 0:64].set(p['W_g2'])
    u2 = _bn_next(ag1, degp, p['b_g1'], p['bn1_w'], p['bn1_b'],
                  wg2p, jnp.zeros((128,), jnp.float32), scale_dis=True)
    ag2 = _sc_aggregate(u2, row, col)

    wp = jnp.zeros((64, 128), jnp.float32)
    wp = wp.at[:, 0:2].set(p['W_out'][:64])
    wp = wp.at[:, 64:66].set(p['W_out'][64:])
    bp = jnp.zeros((128,), jnp.float32).at[0:2].set(p['b_out'])
    zpack = _bn_next(ag2, degp, p['b_g2'], p['bn2_w'], p['bn2_b'],
                     wp, bp, scale_dis=False)
    zr = zpack[:, 0:2].reshape(-1)
    zc = zpack[:, 64:66].reshape(-1)
    o0, o1 = _sc_edge_out(zr, zc, row, col)
    return jnp.stack([o0, o1], axis=1)


# speedup vs baseline: 14.5355x; 1.0683x over previous
import functools

import jax
import jax.numpy as jnp
from jax import lax
from jax.experimental import pallas as pl
from jax.experimental.pallas import tpu as pltpu
from jax.experimental.pallas import tpu_sc as plsc

_NC = 2
_NS = 16
_NW = _NC * _NS


def _mm_bias_body(x_ref, w_ref, b_ref, o_ref):
    o_ref[...] = (
        jnp.dot(x_ref[...], w_ref[...], preferred_element_type=jnp.float32)
        + b_ref[...]
    )


def _mm_bias(x, w, b, bm=1000):
    m, k = x.shape
    n = w.shape[1]
    return pl.pallas_call(
        _mm_bias_body,
        grid=(m // bm,),
        in_specs=[
            pl.BlockSpec((bm, k), lambda i: (i, 0)),
            pl.BlockSpec((k, n), lambda i: (0, 0)),
            pl.BlockSpec((1, n), lambda i: (0, 0)),
        ],
        out_specs=pl.BlockSpec((bm, n), lambda i: (i, 0)),
        out_shape=jax.ShapeDtypeStruct((m, n), jnp.float32),
    )(x, w, b.reshape(1, -1))


def _mm2_bias_body(xa_ref, xb_ref, wa_ref, wb_ref, b_ref, o_ref):
    o_ref[...] = (
        jnp.dot(xa_ref[...], wa_ref[...], preferred_element_type=jnp.float32)
        + jnp.dot(xb_ref[...], wb_ref[...], preferred_element_type=jnp.float32)
        + b_ref[...]
    )


def _mm2_bias(xa, xb, wa, wb, b, bm=1000):
    m, ka = xa.shape
    n = wa.shape[1]
    kb = xb.shape[1]
    return pl.pallas_call(
        _mm2_bias_body,
        grid=(m // bm,),
        in_specs=[
            pl.BlockSpec((bm, ka), lambda i: (i, 0)),
            pl.BlockSpec((bm, kb), lambda i: (i, 0)),
            pl.BlockSpec((ka, n), lambda i: (0, 0)),
            pl.BlockSpec((kb, n), lambda i: (0, 0)),
            pl.BlockSpec((1, n), lambda i: (0, 0)),
        ],
        out_specs=pl.BlockSpec((bm, n), lambda i: (i, 0)),
        out_shape=jax.ShapeDtypeStruct((m, n), jnp.float32),
    )(xa, xb, wa, wb, b.reshape(1, -1))


def _lstm_pair_body(chunk, g, pf_ref, pb_ref, wf_ref, wb_ref,
                    of_ref, ob_ref, st_ref):
    i = pl.program_id(0)

    @pl.when(i == 0)
    def _():
        st_ref[...] = jnp.zeros_like(st_ref)

    wf = wf_ref[...]
    wb = wb_ref[...]

    def gates(gv, c):
        ig = jax.nn.sigmoid(gv[:, 0:128])
        fg = jax.nn.sigmoid(gv[:, 128:256])
        gg = jnp.tanh(gv[:, 256:384])
        og = jax.nn.sigmoid(gv[:, 384:512])
        c = fg * c + ig * gg
        h = og * jnp.tanh(c)
        return h, c

    def step(t, carry):
        hf, cf, hb, cb = carry
        gf = jnp.dot(hf.astype(jnp.bfloat16), wf,
                     preferred_element_type=jnp.float32) \
            + pf_ref[pl.ds(t, 1), :]
        gb = jnp.dot(hb.astype(jnp.bfloat16), wb,
                     preferred_element_type=jnp.float32) \
            + pb_ref[pl.ds(chunk - 1 - t, 1), :]
        hf, cf = gates(gf, cf)
        hb, cb = gates(gb, cb)
        of_ref[pl.ds(t, 1), :] = hf
        ob_ref[pl.ds(chunk - 1 - t, 1), :] = hb
        return hf, cf, hb, cb

    init = (st_ref[0:1, :], st_ref[1:2, :], st_ref[2:3, :], st_ref[3:4, :])
    hf, cf, hb, cb = lax.fori_loop(0, chunk, step, init)
    st_ref[0:1, :] = hf
    st_ref[1:2, :] = cf
    st_ref[2:3, :] = hb
    st_ref[3:4, :] = cb


def _lstm_pair(p_all, whh_f_t, whh_b_t, chunk=1000):
    t_len = p_all.shape[0]
    g = t_len // chunk
    body = functools.partial(_lstm_pair_body, chunk, g)
    return pl.pallas_call(
        body,
        grid=(g,),
        in_specs=[
            pl.BlockSpec((chunk, 512), lambda i: (i, 0)),
            pl.BlockSpec((chunk, 512), lambda i, _g=g: (_g - 1 - i, 1)),
            pl.BlockSpec((128, 512), lambda i: (0, 0)),
            pl.BlockSpec((128, 512), lambda i: (0, 0)),
        ],
        out_specs=[
            pl.BlockSpec((chunk, 128), lambda i: (i, 0)),
            pl.BlockSpec((chunk, 128), lambda i, _g=g: (_g - 1 - i, 0)),
        ],
        out_shape=[
            jax.ShapeDtypeStruct((t_len, 128), jnp.float32),
            jax.ShapeDtypeStruct((t_len, 128), jnp.float32),
        ],
        scratch_shapes=[pltpu.VMEM((8, 128), jnp.float32)],
    )(p_all, p_all, whh_f_t.astype(jnp.bfloat16),
      whh_b_t.astype(jnp.bfloat16))


def _dis_from_degp(dp_ref):
    deg = dp_ref[0] + dp_ref[1]
    return jnp.where(deg > 0, lax.rsqrt(jnp.maximum(deg, 1e-12)), 0.0)


def _u1_body(f1_ref, b1_ref, dp_ref, wa_ref, wb_ref, o_ref):
    dis = _dis_from_degp(dp_ref)
    h = (jnp.dot(f1_ref[...], wa_ref[...], preferred_element_type=jnp.float32)
         + jnp.dot(b1_ref[...], wb_ref[...], preferred_element_type=jnp.float32))
    o_ref[...] = h * dis


def _u1(f1, b1, degp, wa, wb):
    n = f1.shape[0]
    d = wa.shape[1]
    return pl.pallas_call(
        _u1_body,
        out_shape=jax.ShapeDtypeStruct((n, d), jnp.float32),
    )(f1, b1, degp, wa, wb)


def _bn_next_body(d_in, ap_ref, dp_ref, bg_ref, bnw_ref, bnb_ref, w_ref,
                  bo_ref, o_ref):
    dis = _dis_from_degp(dp_ref)
    gv = (ap_ref[0, :, 0:d_in] + ap_ref[1, :, 0:d_in]) * dis + bg_ref[...]
    m = jnp.mean(gv, axis=0, keepdims=True)
    dv = gv - m
    v = jnp.mean(dv * dv, axis=0, keepdims=True)
    gn = dv * lax.rsqrt(v + 1e-5) * bnw_ref[...] + bnb_ref[...]
    r = jnp.maximum(gn, 0.0)
    o_ref[...] = (
        jnp.dot(r, w_ref[...], preferred_element_type=jnp.float32)
        + bo_ref[...]
    )


def _bn_next(ap, degp, bg, bnw, bnb, w, bo, scale_dis):
    n = ap.shape[1]
    d = w.shape[1]
    d_in = bg.shape[0]
    if scale_dis:
        body = functools.partial(_bn_next_scaled_body, d_in)
    else:
        body = functools.partial(_bn_next_body, d_in)
    return pl.pallas_call(
        body,
        out_shape=jax.ShapeDtypeStruct((n, d), jnp.float32),
    )(ap, degp, bg.reshape(1, -1), bnw.reshape(1, -1), bnb.reshape(1, -1),
      w, bo.reshape(1, -1))


def _bn_next_scaled_body(d_in, ap_ref, dp_ref, bg_ref, bnw_ref, bnb_ref,
                         w_ref, bo_ref, o_ref):
    dis = _dis_from_degp(dp_ref)
    gv = (ap_ref[0, :, 0:d_in] + ap_ref[1, :, 0:d_in]) * dis + bg_ref[...]
    m = jnp.mean(gv, axis=0, keepdims=True)
    dv = gv - m
    v = jnp.mean(dv * dv, axis=0, keepdims=True)
    gn = dv * lax.rsqrt(v + 1e-5) * bnw_ref[...] + bnb_ref[...]
    r = jnp.maximum(gn, 0.0)
    o_ref[...] = (
        jnp.dot(r, w_ref[...], preferred_element_type=jnp.float32)
        + bo_ref[...]
    ) * dis


def _sc_mesh():
    return plsc.VectorSubcoreMesh(core_axis_name="c", subcore_axis_name="s",
                                  num_cores=_NC, num_subcores=_NS)


def _row_split(n_nodes):
    per = (n_nodes // (8 * _NS)) * 8
    tail = n_nodes - _NS * per
    return per, tail


def _zero_shared(sh_ref, z_hbm, n_nodes, sid):
    per, tail = _row_split(n_nodes)
    base = sid * per
    nfull = per // 128
    rem = per - nfull * 128
    for k in range(nfull):
        pltpu.sync_copy(z_hbm, sh_ref.at[pl.ds(base + k * 128, 128)])
    if rem:
        pltpu.sync_copy(z_hbm.at[pl.ds(0, rem)],
                        sh_ref.at[pl.ds(base + nfull * 128, rem)])
    if tail:
        @pl.when(sid == 0)
        def _():
            pltpu.sync_copy(z_hbm.at[pl.ds(0, tail)],
                            sh_ref.at[pl.ds(_NS * per, tail)])


def _writeback_shared(sh_ref, out_hbm, n_nodes, cid, sid):
    per, tail = _row_split(n_nodes)
    base = sid * per
    pltpu.sync_copy(sh_ref.at[pl.ds(base, per)],
                    out_hbm.at[cid, pl.ds(base, per)])
    if tail:
        @pl.when(sid == 0)
        def _():
            pltpu.sync_copy(sh_ref.at[pl.ds(_NS * per, tail)],
                            out_hbm.at[cid, pl.ds(_NS * per, tail)])


def _deg_body(n_pad, ept, ch, col_hbm, out_hbm, deg_v, col_v, sh_ref,
              buf_v, acc_v):
    cid = lax.axis_index("c")
    sid = lax.axis_index("s")
    wid = sid * _NC + cid
    zero16 = jnp.zeros((16,), jnp.float32)
    one16 = jnp.ones((16,), jnp.float32)

    def zero_body(i, _):
        deg_v[pl.ds(i * 16, 16)] = zero16
        return 0

    lax.fori_loop(0, n_pad // 16, zero_body, 0)

    nch = ept // ch

    def chunk_body(c, _):
        pltpu.sync_copy(col_hbm.at[pl.ds(wid * ept + c * ch, ch)], col_v)

        def it(k, _2):
            idx16 = col_v[pl.ds(k * 16, 16)]
            plsc.addupdate_scatter(deg_v, [idx16], one16)
            return 0

        lax.fori_loop(0, ch // 16, it, 0)
        return 0

    lax.fori_loop(0, nch, chunk_body, 0)

    pltpu.sync_copy(deg_v, sh_ref.at[sid])
    plsc.subcore_barrier()
    rows = n_pad // _NS
    base = sid * rows

    def zero2(i, _):
        acc_v[pl.ds(i * 16, 16)] = zero16
        return 0

    lax.fori_loop(0, rows // 16, zero2, 0)
    for j in range(_NS):
        pltpu.sync_copy(sh_ref.at[j, pl.ds(base, rows)], buf_v)

        def addb(i, _):
            acc_v[pl.ds(i * 16, 16)] = (acc_v[pl.ds(i * 16, 16)]
                                        + buf_v[pl.ds(i * 16, 16)])
            return 0

        lax.fori_loop(0, rows // 16, addb, 0)
    pltpu.sync_copy(acc_v, out_hbm.at[cid, pl.ds(base, rows)])


def _sc_degree(col, n_nodes):
    e = col.shape[0]
    ept = e // _NW
    ch = 4000
    n_pad = ((n_nodes + 2047) // 2048) * 2048
    body = functools.partial(_deg_body, n_pad, ept, ch)
    f = pl.kernel(
        body,
        out_type=jax.ShapeDtypeStruct((_NC, n_pad), jnp.float32),
        mesh=_sc_mesh(),
        compiler_params=pltpu.CompilerParams(needs_layout_passes=False),
        scratch_types=[
            pltpu.VMEM((n_pad,), jnp.float32),
            pltpu.VMEM((ch,), jnp.int32),
            pltpu.VMEM_SHARED((_NS, n_pad), jnp.float32),
            pltpu.VMEM((n_pad // _NS,), jnp.float32),
            pltpu.VMEM((n_pad // _NS,), jnp.float32),
        ],
    )
    return f(col)


_NBUF = 2
_CHB = 16


def _agg_body(n_nodes, d, ept, u_hbm, row_hbm, col_hbm, z_hbm, out_hbm,
              sh_ref, ri_c, ci_c, rows0, rows1, cb0, cb1, ic_r,
              sem0, sem1):
    cid = lax.axis_index("c")
    sid = lax.axis_index("s")
    wid = sid * _NC + cid
    rows_bufs = [rows0, rows1]
    cb_bufs = [cb0, cb1]
    sems = [sem0, sem1]
    _zero_shared(sh_ref, z_hbm, n_nodes, sid)
    plsc.subcore_barrier()

    ebase = wid * ept
    ch = _CHB * 128
    nb = ept // 128
    rem = ept - nb * 128
    ngroups = nb // _NBUF
    nch_full = nb // _CHB
    g_per_ch = _CHB // _NBUF
    g_tail_start = nch_full * g_per_ch
    tail_edges = ept - nch_full * ch

    def group(g, _):
        @pl.when(jnp.logical_and(g % g_per_ch == 0, g < g_tail_start))
        def _():
            c = g // g_per_ch
            pltpu.sync_copy(row_hbm.at[pl.ds(ebase + c * ch, ch)],
                            ri_c.at[pl.ds(0, ch)])
            pltpu.sync_copy(col_hbm.at[pl.ds(ebase + c * ch, ch)],
                            ci_c.at[pl.ds(0, ch)])

        if tail_edges > 0 and g_tail_start < ngroups + 1:
            @pl.when(g == g_tail_start)
            def _():
                pltpu.sync_copy(
                    row_hbm.at[pl.ds(ebase + nch_full * ch, tail_edges)],
                    ri_c.at[pl.ds(0, tail_edges)])
                pltpu.sync_copy(
                    col_hbm.at[pl.ds(ebase + nch_full * ch, tail_edges)],
                    ci_c.at[pl.ds(0, tail_edges)])

        bl0 = (g % g_per_ch) * _NBUF
        gdescs = []
        for j in range(_NBUF):
            bl = bl0 + j
            for k in range(8):
                cb_bufs[j][pl.ds(k * 16, 16)] = \
                    ci_c[pl.ds(bl * 128 + k * 16, 16)]
            gdescs.append(pltpu.async_copy(
                u_hbm.at[ri_c.at[pl.ds(bl * 128, 128)]],
                rows_bufs[j], sems[j]))
        sdescs = []
        for j in range(_NBUF):
            gdescs[j].wait()
            sdescs.append(pltpu.async_copy(
                rows_bufs[j], sh_ref.at[cb_bufs[j]], sems[j], add=True))
        for j in range(_NBUF):
            sdescs[j].wait()
        return 0

    lax.fori_loop(0, ngroups, group, 0)
    if rem:
        tb = nb - nch_full * _CHB
        off = tb * 128
        for k in range(rem // 16):
            ic_r[pl.ds(k * 16, 16)] = ci_c[pl.ds(off + k * 16, 16)]
        pltpu.async_copy(u_hbm.at[ri_c.at[pl.ds(off, rem)]],
                         rows0.at[pl.ds(0, rem)], sem0).wait()
        pltpu.sync_copy(rows0.at[pl.ds(0, rem)], sh_ref.at[ic_r], add=True)
    plsc.subcore_barrier()
    _writeback_shared(sh_ref, out_hbm, n_nodes, cid, sid)


def _sc_aggregate(u, row, col):
    n_nodes, d = u.shape
    e = row.shape[0]
    ept = e // _NW
    z_blk = jnp.zeros((128, d), jnp.float32)
    ch = _CHB * 128
    body = functools.partial(_agg_body, n_nodes, d, ept)
    f = pl.kernel(
        body,
        out_type=jax.ShapeDtypeStruct((_NC, n_nodes, d), jnp.float32),
        mesh=_sc_mesh(),
        scratch_types=[
            pltpu.VMEM_SHARED((n_nodes, d), jnp.float32),
            pltpu.VMEM((ch,), jnp.int32),
            pltpu.VMEM((ch,), jnp.int32),
            pltpu.VMEM((128, d), jnp.float32),
            pltpu.VMEM((128, d), jnp.float32),
            pltpu.VMEM((128,), jnp.int32),
            pltpu.VMEM((128,), jnp.int32),
            pltpu.VMEM((32,), jnp.int32),
            pltpu.SemaphoreType.DMA,
            pltpu.SemaphoreType.DMA,
        ],
    )
    return f(u, row, col, z_blk)


def _edge_out_body(n_nodes, ept, ch, zr_hbm, zc_hbm, row_hbm, col_hbm,
                   o0_hbm, o1_hbm, zr_v, zc_v, row_v, col_v, o0_v, o1_v):
    cid = lax.axis_index("c")
    sid = lax.axis_index("s")
    wid = sid * _NC + cid
    ebase = wid * ept
    pltpu.sync_copy(zr_hbm, zr_v)
    pltpu.sync_copy(zc_hbm, zc_v)

    nch = ept // ch

    def chunk_body(c, _):
        cbase = ebase + c * ch
        pltpu.sync_copy(row_hbm.at[pl.ds(cbase, ch)], row_v)
        pltpu.sync_copy(col_hbm.at[pl.ds(cbase, ch)], col_v)

        def it(k, _2):
            r2 = row_v[pl.ds(k * 16, 16)] * 2
            c2 = col_v[pl.ds(k * 16, 16)] * 2
            a0 = plsc.load_gather(zr_v, [r2])
            a1 = plsc.load_gather(zr_v, [r2 + 1])
            b0 = plsc.load_gather(zc_v, [c2])
            b1 = plsc.load_gather(zc_v, [c2 + 1])
            o0_v[pl.ds(k * 16, 16)] = a0 + b0
            o1_v[pl.ds(k * 16, 16)] = a1 + b1
            return 0

        lax.fori_loop(0, ch // 16, it, 0)
        pltpu.sync_copy(o0_v, o0_hbm.at[pl.ds(cbase, ch)])
        pltpu.sync_copy(o1_v, o1_hbm.at[pl.ds(cbase, ch)])
        return 0

    lax.fori_loop(0, nch, chunk_body, 0)


def _sc_edge_out(zr, zc, row, col):
    table_len = zr.shape[0]
    e = row.shape[0]
    ept = e // _NW
    ch = 4000
    body = functools.partial(_edge_out_body, table_len, ept, ch)
    f = pl.kernel(
        body,
        out_type=(
            jax.ShapeDtypeStruct((e,), jnp.float32),
            jax.ShapeDtypeStruct((e,), jnp.float32),
        ),
        mesh=_sc_mesh(),
        compiler_params=pltpu.CompilerParams(needs_layout_passes=False),
        scratch_types=[
            pltpu.VMEM((table_len,), jnp.float32),
            pltpu.VMEM((table_len,), jnp.float32),
            pltpu.VMEM((ch,), jnp.int32),
            pltpu.VMEM((ch,), jnp.int32),
            pltpu.VMEM((ch,), jnp.float32),
            pltpu.VMEM((ch,), jnp.float32),
        ],
    )
    return f(zr, zc, row, col)


def kernel(x, edge_index, params):
    p = params
    n_nodes = x.shape[0]
    row = edge_index[0]
    col = edge_index[1]

    w0 = jnp.concatenate([p['Wih_l0f'].T, p['Wih_l0b'].T], axis=1)
    bias0 = jnp.concatenate([p['bih_l0f'] + p['bhh_l0f'],
                             p['bih_l0b'] + p['bhh_l0b']])
    p0 = _mm_bias(x, w0, bias0)
    f0, b0 = _lstm_pair(p0, p['Whh_l0f'].T, p['Whh_l0b'].T)

    w1 = jnp.concatenate([p['Wih_l1f'].T, p['Wih_l1b'].T], axis=1)
    bias1 = jnp.concatenate([p['bih_l1f'] + p['bhh_l1f'],
                             p['bih_l1b'] + p['bhh_l1b']])
    p1 = _mm2_bias(f0, b0, w1[:128], w1[128:], bias1)
    f1, b1 = _lstm_pair(p1, p['Whh_l1f'].T, p['Whh_l1b'].T)

    degp = _sc_degree(col, n_nodes)
    degp = degp[:, :n_nodes, None]

    u1 = _u1(f1, b1, degp, p['W_g1'][:128], p['W_g1'][128:])
    ag1 = _sc_aggregate(u1, row, col)
    wg2p = jnp.zeros((128, 128), jnp.float32).at[:, 0:64].set(p['W_g2'])
    u2 = _bn_next(ag1, degp, p['b_g1'], p['bn1_w'], p['bn1_b'],
                  wg2p, jnp.zeros((128,), jnp.float32), scale_dis=True)
    ag2 = _sc_aggregate(u2, row, col)

    wp = jnp.zeros((64, 128), jnp.float32)
    wp = wp.at[:, 0:2].set(p['W_out'][:64])
    wp = wp.at[:, 64:66].set(p['W_out'][64:])
    bp = jnp.zeros((128,), jnp.float32).at[0:2].set(p['b_out'])
    zpack = _bn_next(ag2, degp, p['b_g2'], p['bn2_w'], p['bn2_b'],
                     wp, bp, scale_dis=False)
    zr = zpack[:, 0:2].reshape(-1)
    zc = zpack[:, 64:66].reshape(-1)
    o0, o1 = _sc_edge_out(zr, zc, row, col)
    return jnp.stack([o0, o1], axis=1)
